# Initial kernel scaffold; baseline (speedup 1.0000x reference)
import jax
import jax.numpy as jnp
from jax.experimental import pallas as pl


def kernel(data, org_edge_index, pastmrr, emb, W, lin_b, att_i, att_j, bn_scale, bn_shift, W_ih, W_hh, b_ih, b_hh, mlp_W, mlp_b):
    raise NotImplementedError("write your pallas kernel here")


# speedup vs baseline: 1.7247x; 1.7247x over previous
import functools
import jax
import jax.numpy as jnp
from jax import lax
from jax.experimental import pallas as pl
from jax.experimental.pallas import tpu as pltpu
from jax.experimental.pallas import tpu_sc as plsc

N = 10000
D = 64
K = 12
PAST = 4
B = 8
NW = 32
T = 320
NP = NW * T
G = 8
NCH = T // G
R1 = 200
KC = 3200
GRU_IN = (N // PAST) * D


def _topk_body(emb_ref, eblk_ref, atti_ref, attj_ref, topk_ref, ae_ref, de_ref):
    emb = emb_ref[...]
    ss = jnp.sum(emb * emb, axis=1, keepdims=True)
    wn = emb / (jnp.sqrt(ss) + 1e-8)
    eblk = eblk_ref[...]
    sblk = jnp.sum(eblk * eblk, axis=1, keepdims=True)
    blk = eblk / (jnp.sqrt(sblk) + 1e-8)
    scores = lax.dot_general(blk, wn, (((1,), (1,)), ((), ())),
                             preferred_element_type=jnp.float32)
    ae_ref[0, 0, :] = eblk @ attj_ref[0, D:]
    de_ref[0, 0, :] = eblk @ atti_ref[0, D:]
    iota = lax.broadcasted_iota(jnp.int32, (R1, N), 1)
    s = scores
    cols = []
    for _ in range(K):
        m = jnp.max(s, axis=1, keepdims=True)
        idx = jnp.min(jnp.where(s == m, iota, N), axis=1, keepdims=True)
        cols.append(idx)
        s = jnp.where(iota == idx, -3.0e38, s)
    topk_ref[0] = jnp.concatenate(cols, axis=1)


def _proj_body(x_ref, w_ref, b_ref, atti_ref, attj_ref, xp_ref, ax_ref, dx_ref):
    xp = jnp.dot(x_ref[...], w_ref[...],
                 preferred_element_type=jnp.float32) + b_ref[0]
    xp_ref[...] = xp
    ax_ref[0, 0, :] = xp @ attj_ref[0, :D]
    dx_ref[0, 0, :] = xp @ atti_ref[0, :D]


def _sc_exp(x):
    x = jnp.maximum(x, -87.0)
    y = x * 1.4426950408889634
    n = (y - 0.5).astype(jnp.int32)
    nf = n.astype(jnp.float32)
    t = x - nf * 0.6931471805599453
    p = jnp.full_like(t, 1.0 / 720.0)
    for c in (1.0 / 120.0, 1.0 / 24.0, 1.0 / 6.0, 0.5, 1.0, 1.0):
        p = p * t + c
    two_n = plsc.bitcast((n + 127) << 23, jnp.float32)
    return p * two_n


def _sc_body(xp_hbm, g_hbm, topk_hbm, c_hbm, emb_hbm, bn_hbm, gdn_hbm,
             topk_v, c_v, emb_v, bn_v, idx_v, g_v, rows_v, out_v,
             sem_a, sem_b):
    wid = lax.axis_index("c") * 16 + lax.axis_index("s")
    base = wid * T
    lane = lax.iota(jnp.int32, 16)
    pltpu.sync_copy(topk_hbm.at[pl.ds(base * 16, T * 16)], topk_v)
    for b in range(B):
        pltpu.sync_copy(c_hbm.at[pl.ds(b * NP + base, T)],
                        c_v.at[pl.ds(b * T, T)])
    pltpu.sync_copy(emb_hbm.at[pl.ds(base * D, T * D)], emb_v)
    pltpu.sync_copy(bn_hbm, bn_v)
    def bbody(b, carry0):
        def chunk(ch, carry):
            for p in range(G):
                j = ch * G + p
                idxv = plsc.load_gather(topk_v, [j * 16 + lane])
                gidx = b * N + jnp.where(lane < K, idxv, 0)
                idx_v[pl.ds(p * 16, 16)] = gidx
            cp_r = pltpu.make_async_copy(xp_hbm.at[idx_v], rows_v, sem_a)
            cp_g = pltpu.make_async_copy(g_hbm.at[idx_v], g_v, sem_b)
            cp_r.start()
            cp_g.start()
            cp_r.wait()
            cp_g.wait()
            for p in range(G):
                j = ch * G + p
                g16 = g_v[pl.ds(p * 16, 16)]
                c16 = plsc.load_gather(c_v, [jnp.full((16,), b * T, jnp.int32) + j])
                sc = c16 + g16
                sc = jnp.where(sc > 0, sc, 0.2 * sc)
                sc = jnp.where(lane < K, sc, -3.0e38)
                m = jnp.max(sc)
                e = _sc_exp(sc - m)
                e = jnp.where(lane < K, e, 0.0)
                sv = jnp.zeros((16,), jnp.float32) + jnp.sum(e)
                rv = 1.0 / sv
                rv = rv * (2.0 - sv * rv)
                alpha = e * rv
                acc = [jnp.zeros((16,), jnp.float32) for _ in range(4)]
                for k in range(K):
                    ak = jnp.sum(jnp.where(lane == k, alpha, 0.0))
                    r = p * 16 + k
                    for c in range(4):
                        acc[c] = acc[c] + ak * rows_v[r, pl.ds(c * 16, 16)]
                for c in range(4):
                    cl = j * D + c * 16 + lane
                    h = (acc[c] * bn_v[pl.ds(c * 16, 16)]
                         + bn_v[pl.ds(D + c * 16, 16)])
                    h = jnp.maximum(h, 0.0)
                    o = h * plsc.load_gather(emb_v, [cl])
                    plsc.store_scatter(out_v, [cl], o)
            return carry
        lax.fori_loop(0, NCH, chunk, 0)
        pltpu.sync_copy(out_v, gdn_hbm.at[pl.ds((b * NP + base) * D, T * D)])
        return carry0
    lax.fori_loop(0, B, bbody, 0)


def _gru_body(x_ref, wih_ref, whh_ref, bih_ref, bhh_ref, pm_ref, mw_ref,
              mb_ref, out_ref, acc_ref):
    k = pl.program_id(0)

    @pl.when(k == 0)
    def _():
        acc_ref[...] = jnp.zeros_like(acc_ref)

    acc_ref[...] += lax.dot_general(
        x_ref[...], wih_ref[...], (((1,), (1,)), ((), ())),
        preferred_element_type=jnp.float32)

    @pl.when(k == pl.num_programs(0) - 1)
    def _():
        gi = acc_ref[...] + bih_ref[0]
        whh = whh_ref[...]
        bhh = bhh_ref[0]
        h = jnp.zeros((B, D), jnp.float32)
        for t in range(PAST):
            git = gi[t * B:(t + 1) * B, :]
            gh = lax.dot_general(h, whh, (((1,), (1,)), ((), ())),
                                 preferred_element_type=jnp.float32) + bhh
            r = jax.nn.sigmoid(git[:, :D] + gh[:, :D])
            z = jax.nn.sigmoid(git[:, D:2 * D] + gh[:, D:2 * D])
            n = jnp.tanh(git[:, 2 * D:] + r * gh[:, 2 * D:])
            h = (1.0 - z) * n + z * h
        gru = jnp.clip(h, 0.0, 1.0)
        wv = mw_ref[0]
        o = gru @ wv[:D] + pm_ref[...] @ wv[D:] + mb_ref[0, 0]
        out_ref[...] = jnp.maximum(o, 0.0)[:, None]


def kernel(data, org_edge_index, pastmrr, emb, W, lin_b, att_i, att_j,
           bn_scale, bn_shift, W_ih, W_hh, b_ih, b_hh, mlp_W, mlp_b):
    atti2 = att_i.reshape(1, 2 * D)
    attj2 = att_j.reshape(1, 2 * D)

    nblk = N // R1
    topk3, ae3, de3 = pl.pallas_call(
        _topk_body,
        grid=(nblk,),
        in_specs=[
            pl.BlockSpec((N, D), lambda i: (0, 0)),
            pl.BlockSpec((R1, D), lambda i: (i, 0)),
            pl.BlockSpec((1, 2 * D), lambda i: (0, 0)),
            pl.BlockSpec((1, 2 * D), lambda i: (0, 0)),
        ],
        out_specs=[
            pl.BlockSpec((1, R1, K), lambda i: (i, 0, 0)),
            pl.BlockSpec((1, 1, R1), lambda i: (i, 0, 0)),
            pl.BlockSpec((1, 1, R1), lambda i: (i, 0, 0)),
        ],
        out_shape=[
            jax.ShapeDtypeStruct((nblk, R1, K), jnp.int32),
            jax.ShapeDtypeStruct((nblk, 1, R1), jnp.float32),
            jax.ShapeDtypeStruct((nblk, 1, R1), jnp.float32),
        ],
    )(emb, emb, atti2, attj2)
    topk_idx = topk3.reshape(N, K)
    ae = ae3.reshape(N)
    de = de3.reshape(N)

    M = 1000
    data_flat = data.reshape(B * N, data.shape[-1])
    nblk2 = (B * N) // M
    xp, ax3, dx3 = pl.pallas_call(
        _proj_body,
        grid=(nblk2,),
        in_specs=[
            pl.BlockSpec((M, data.shape[-1]), lambda i: (i, 0)),
            pl.BlockSpec((data.shape[-1], D), lambda i: (0, 0)),
            pl.BlockSpec((1, D), lambda i: (0, 0)),
            pl.BlockSpec((1, 2 * D), lambda i: (0, 0)),
            pl.BlockSpec((1, 2 * D), lambda i: (0, 0)),
        ],
        out_specs=[
            pl.BlockSpec((M, D), lambda i: (i, 0)),
            pl.BlockSpec((1, 1, M), lambda i: (i, 0, 0)),
            pl.BlockSpec((1, 1, M), lambda i: (i, 0, 0)),
        ],
        out_shape=[
            jax.ShapeDtypeStruct((B * N, D), jnp.float32),
            jax.ShapeDtypeStruct((nblk2, 1, M), jnp.float32),
            jax.ShapeDtypeStruct((nblk2, 1, M), jnp.float32),
        ],
    )(data_flat, W, lin_b.reshape(1, D), atti2, attj2)
    ax = ax3.reshape(B, N)
    dx = dx3.reshape(B, N)

    g_flat = (ax + ae[None, :]).reshape(B * N)
    c_pad = jnp.pad(dx + de[None, :], ((0, 0), (0, NP - N))).reshape(B * NP)
    topk_pad = jnp.pad(topk_idx, ((0, NP - N), (0, 16 - K)))
    emb_pad = jnp.pad(emb, ((0, NP - N), (0, 0)))
    bn2 = jnp.stack([bn_scale, bn_shift], axis=0)

    mesh = plsc.VectorSubcoreMesh(core_axis_name="c", subcore_axis_name="s",
                                  num_cores=2, num_subcores=16)
    gdn_flat = pl.kernel(
        _sc_body,
        out_type=jax.ShapeDtypeStruct((B * NP * D,), jnp.float32),
        mesh=mesh,
        compiler_params=pltpu.CompilerParams(needs_layout_passes=False,
                                             use_tc_tiling_on_sc=False),
        scratch_types=[
            pltpu.VMEM((T * 16,), jnp.int32),
            pltpu.VMEM((B * T,), jnp.float32),
            pltpu.VMEM((T * D,), jnp.float32),
            pltpu.VMEM((2 * D,), jnp.float32),
            pltpu.VMEM((G * 16,), jnp.int32),
            pltpu.VMEM((G * 16,), jnp.float32),
            pltpu.VMEM((G * 16, D), jnp.float32),
            pltpu.VMEM((T * D,), jnp.float32),
            pltpu.SemaphoreType.DMA,
            pltpu.SemaphoreType.DMA,
        ],
    )(xp, g_flat, topk_pad.reshape(NP * 16), c_pad,
      emb_pad.reshape(NP * D), bn2.reshape(2 * D))
    gdn_pad = gdn_flat.reshape(B, NP, D)

    x_all = gdn_pad[:, :N, :].reshape(B, PAST, GRU_IN)
    x_all = x_all.transpose(1, 0, 2).reshape(PAST * B, GRU_IN)
    nk = GRU_IN // KC
    out = pl.pallas_call(
        _gru_body,
        grid=(nk,),
        in_specs=[
            pl.BlockSpec((PAST * B, KC), lambda k: (0, k)),
            pl.BlockSpec((3 * D, KC), lambda k: (0, k)),
            pl.BlockSpec((3 * D, D), lambda k: (0, 0)),
            pl.BlockSpec((1, 3 * D), lambda k: (0, 0)),
            pl.BlockSpec((1, 3 * D), lambda k: (0, 0)),
            pl.BlockSpec((B, PAST), lambda k: (0, 0)),
            pl.BlockSpec((1, D + PAST), lambda k: (0, 0)),
            pl.BlockSpec((1, 1), lambda k: (0, 0)),
        ],
        out_specs=pl.BlockSpec((B, 1), lambda k: (0, 0)),
        out_shape=jax.ShapeDtypeStruct((B, 1), jnp.float32),
        scratch_shapes=[pltpu.VMEM((PAST * B, 3 * D), jnp.float32)],
    )(x_all, W_ih, W_hh, b_ih.reshape(1, 3 * D), b_hh.reshape(1, 3 * D),
      pastmrr, mlp_W.reshape(1, D + PAST), mlp_b.reshape(1, 1))
    return out
